# Initial kernel scaffold; baseline (speedup 1.0000x reference)
import jax
import jax.numpy as jnp
from jax.experimental import pallas as pl


def kernel(observation, edge_index, W1, b1, W2, b2, Wg, bg, Wgd, bgd, Wp1, bp1, Wp2, bp2, Wpi, bpi):
    raise NotImplementedError("write your pallas kernel here")


# speedup vs baseline: 20.4252x; 20.4252x over previous
import numpy as np

import jax
import jax.numpy as jnp
from jax import lax
from jax.experimental import pallas as pl
from jax.experimental.pallas import tpu as pltpu
from jax.experimental.pallas import tpu_sc as plsc

N = 10000
FD = 128
A = 2
E = 160000
H = 32

L = 16
NPAD = 10240
NPT = 640
MCHUNK = 1000
EROWS = E // MCHUNK
MCH = EROWS // 32
DCH = EROWS // 16

_RSQRT_LUT = (1.0 / np.sqrt(np.arange(1, E + 2, dtype=np.float64))).astype(np.float32)


def _sc_body(ei_hbm, xw_hbm, lut_hbm, acc_out, dinv_out,
             y_sh, acc_sh, deg_sh, src_v, dst_v, rows0, rows1,
             ones_v, dinv_v, degidx_v, semA, semB):
    cid = lax.axis_index("c")
    sid = lax.axis_index("s")
    nbase = sid * NPT

    cpA = pltpu.async_copy(ei_hbm.at[1, pl.ds(sid * DCH, MCH), :], src_v, semA)
    cpB = pltpu.async_copy(ei_hbm.at[1, pl.ds(sid * DCH + MCH, MCH), :], dst_v, semB)

    zero16 = jnp.zeros((L,), jnp.float32)
    one16 = jnp.ones((L,), jnp.float32)

    @pl.loop(0, 1024 // L)
    def _(i):
        ones_v[pl.ds(i * L, L)] = one16

    @pl.loop(0, NPT // L)
    def _(i):
        dinv_v[pl.ds(i * L, L)] = zero16

    @pl.loop(0, NPT)
    def _(i):
        rows0[i, 0:16] = zero16
        rows0[i, 16:32] = zero16

    pltpu.sync_copy(dinv_v, deg_sh.at[pl.ds(nbase, NPT)])
    pltpu.sync_copy(rows0.at[pl.ds(0, NPT), :], acc_sh.at[pl.ds(nbase, NPT), :])
    cpA.wait()
    cpB.wait()
    plsc.subcore_barrier()

    degs = []
    for j in range(MCH):
        degs.append(pltpu.async_copy(
            ones_v.at[pl.ds(0, MCHUNK)], deg_sh.at[src_v.at[j]], semA, add=True))
    for j in range(MCH):
        degs.append(pltpu.async_copy(
            ones_v.at[pl.ds(0, MCHUNK)], deg_sh.at[dst_v.at[j]], semB, add=True))

    pltpu.sync_copy(xw_hbm.at[pl.ds(nbase, NPT), :], rows1.at[pl.ds(0, NPT), :])

    for d in degs:
        d.wait()
    plsc.subcore_barrier()

    pltpu.sync_copy(deg_sh.at[pl.ds(nbase, NPT)], dinv_v)

    @pl.loop(0, NPT // L)
    def _(i):
        degidx_v[pl.ds(i * L, L)] = dinv_v[pl.ds(i * L, L)].astype(jnp.int32)

    pltpu.async_copy(lut_hbm.at[degidx_v], dinv_v, semA).wait()

    @pl.when(cid == 0)
    def _():
        pltpu.sync_copy(dinv_v, dinv_out.at[pl.ds(nbase, NPT)])

    @pl.loop(0, NPT)
    def _(i):
        d = plsc.load_gather(dinv_v, [jnp.full((L,), i, jnp.int32)])
        rows1[i, 0:16] = rows1[i, 0:16] * d
        rows1[i, 16:32] = rows1[i, 16:32] * d

    pltpu.sync_copy(rows1.at[pl.ds(0, NPT), :], y_sh.at[pl.ds(nbase, NPT), :])
    plsc.subcore_barrier()

    rbase = cid * (EROWS // 2) + sid * MCH
    cpA = pltpu.async_copy(ei_hbm.at[0, pl.ds(rbase, MCH), :], src_v, semA)
    cpB = pltpu.async_copy(ei_hbm.at[1, pl.ds(rbase, MCH), :], dst_v, semB)
    cpA.wait()
    cpB.wait()

    bufs = (rows0, rows1)
    sems = (semA, semB)
    descs = [None] * MCH
    descs[0] = pltpu.async_copy(y_sh.at[src_v.at[0]], rows0, semA)
    for j in range(MCH):
        descs[j].wait()
        if j + 1 < MCH:
            descs[j + 1] = pltpu.async_copy(
                y_sh.at[src_v.at[j + 1]], bufs[(j + 1) % 2], sems[(j + 1) % 2])
        pltpu.sync_copy(bufs[j % 2], acc_sh.at[dst_v.at[j]], add=True)

    plsc.subcore_barrier()
    pltpu.sync_copy(acc_sh.at[pl.ds(nbase, NPT), :],
                    acc_out.at[cid, pl.ds(nbase, NPT), :])


_sc_fused = pl.kernel(
    _sc_body,
    out_type=(
        jax.ShapeDtypeStruct((2, NPAD, H), jnp.float32),
        jax.ShapeDtypeStruct((NPAD,), jnp.float32),
    ),
    mesh=plsc.VectorSubcoreMesh(core_axis_name="c", subcore_axis_name="s"),
    scratch_types=[
        pltpu.VMEM_SHARED((NPAD, H), jnp.float32),
        pltpu.VMEM_SHARED((NPAD, H), jnp.float32),
        pltpu.VMEM_SHARED((NPAD,), jnp.float32),
        pltpu.VMEM((MCH, MCHUNK), jnp.int32),
        pltpu.VMEM((MCH, MCHUNK), jnp.int32),
        pltpu.VMEM((MCHUNK, H), jnp.float32),
        pltpu.VMEM((MCHUNK, H), jnp.float32),
        pltpu.VMEM((1024,), jnp.float32),
        pltpu.VMEM((NPT,), jnp.float32),
        pltpu.VMEM((NPT,), jnp.int32),
        pltpu.SemaphoreType.DMA,
        pltpu.SemaphoreType.DMA,
    ],
    compiler_params=pltpu.CompilerParams(
        use_tc_tiling_on_sc=False, needs_layout_passes=False),
)


BLK = 2048


def _tc1_body(obs_ref, w1_ref, b1_ref, w2_ref, b2_ref, wg_ref, x_out, xw_out):
    h = jnp.maximum(
        jnp.dot(obs_ref[...], w1_ref[...], preferred_element_type=jnp.float32)
        + b1_ref[...], 0.0)
    x = jnp.maximum(
        jnp.dot(h, w2_ref[...], preferred_element_type=jnp.float32)
        + b2_ref[...], 0.0)
    x_out[...] = x
    xw_out[...] = jnp.dot(x, wg_ref[...], preferred_element_type=jnp.float32)


def _tc2_body(acc_ref, dinv_ref, xw_ref, x_ref, bg_ref, wgd_ref, bgd_ref,
              wp1a_ref, wp1b_ref, bp1_ref, wp2_ref, bp2_ref, wpi_ref, bpi_ref,
              out_ref):
    accs = acc_ref[0] + acc_ref[1]
    dinv = dinv_ref[...]
    g = dinv * (accs + dinv * xw_ref[...]) + bg_ref[...]
    xg = jnp.maximum(g, 0.0)
    xg = jnp.maximum(
        jnp.dot(xg, wgd_ref[...], preferred_element_type=jnp.float32)
        + bgd_ref[...], 0.0)
    p = jnp.maximum(
        jnp.dot(xg, wp1a_ref[...], preferred_element_type=jnp.float32)
        + jnp.dot(x_ref[...], wp1b_ref[...], preferred_element_type=jnp.float32)
        + bp1_ref[...], 0.0)
    p = jnp.maximum(
        jnp.dot(p, wp2_ref[...], preferred_element_type=jnp.float32)
        + bp2_ref[...], 0.0)
    pi = jnp.dot(p, wpi_ref[...], preferred_element_type=jnp.float32) + bpi_ref[...]
    out_ref[...] = jnp.tanh(pi)


def _full(shape):
    return pl.BlockSpec(shape, lambda i: tuple(0 for _ in shape))


def kernel(observation, edge_index, W1, b1, W2, b2, Wg, bg, Wgd, bgd,
           Wp1, bp1, Wp2, bp2, Wpi, bpi):
    obs_pad = jnp.pad(observation, ((0, NPAD - N), (0, 0)))
    ei3 = edge_index.reshape(2, EROWS, MCHUNK)
    lut = jnp.asarray(_RSQRT_LUT)

    x_enc, xw = pl.pallas_call(
        _tc1_body,
        grid=(NPAD // BLK,),
        in_specs=[
            pl.BlockSpec((BLK, FD), lambda i: (i, 0)),
            _full((FD, H)),
            _full((1, H)),
            _full((H, H)),
            _full((1, H)),
            _full((H, H)),
        ],
        out_specs=[pl.BlockSpec((BLK, H), lambda i: (i, 0))] * 2,
        out_shape=[jax.ShapeDtypeStruct((NPAD, H), jnp.float32)] * 2,
    )(obs_pad, W1, b1.reshape(1, H), W2, b2.reshape(1, H), Wg)

    acc, dinv = _sc_fused(ei3, xw, lut)

    out = pl.pallas_call(
        _tc2_body,
        grid=(NPAD // BLK,),
        in_specs=[
            pl.BlockSpec((2, BLK, H), lambda i: (0, i, 0)),
            pl.BlockSpec((BLK, 1), lambda i: (i, 0)),
            pl.BlockSpec((BLK, H), lambda i: (i, 0)),
            pl.BlockSpec((BLK, H), lambda i: (i, 0)),
            _full((1, H)),
            _full((H, H)),
            _full((1, H)),
            _full((H, H)),
            _full((H, H)),
            _full((1, H)),
            _full((H, H)),
            _full((1, H)),
            _full((H, A)),
            _full((1, A)),
        ],
        out_specs=pl.BlockSpec((BLK, A), lambda i: (i, 0)),
        out_shape=jax.ShapeDtypeStruct((NPAD, A), jnp.float32),
    )(acc, dinv.reshape(NPAD, 1), xw, x_enc, bg.reshape(1, H), Wgd,
      bgd.reshape(1, H), Wp1[:H], Wp1[H:], bp1.reshape(1, H), Wp2,
      bp2.reshape(1, H), Wpi, bpi.reshape(1, A))

    return out[:N]
